# baseline (speedup 1.0000x reference)
import functools

import jax
import jax.numpy as jnp
from jax import lax
from jax.experimental import pallas as pl
from jax.experimental.pallas import tpu as pltpu
from jax.experimental.pallas import tpu_sc as plsc

N = 10000
D = 128
E = 320000
NC = 2
NS = 16
NW = NC * NS
CHUNK = 128
KPT = 80
NCHUNK = NW * KPT
E_PAD = NCHUNK * CHUNK
GCHUNK = 128
GTCH = E_PAD // GCHUNK
NBUF = 2
GSTAGE = 16
F0 = 160
F1 = 0
assert NS * (F0 + F1) == GTCH
ACC_ROWS = 10240
RPT = ACC_ROWS // NS

_MESH = plsc.VectorSubcoreMesh(core_axis_name="c", subcore_axis_name="s")


@functools.partial(
    pl.kernel,
    out_type=jax.ShapeDtypeStruct((NC * ACC_ROWS, D), jnp.float32),
    mesh=_MESH,
    scratch_types=[
        pltpu.VMEM((KPT, CHUNK), jnp.int32),
        pltpu.VMEM((CHUNK, D), jnp.float32),
        pltpu.VMEM((CHUNK, D), jnp.float32),
        pltpu.VMEM_SHARED((ACC_ROWS, D), jnp.float32),
    ],
)
def _sc_degree(dst_hbm, ones_hbm, zeros_hbm, out_hbm,
               idx_v, ones_v, buf_v, deg_sh):
  cid = lax.axis_index("c")
  sid = lax.axis_index("s")
  wid = sid * NC + cid
  pltpu.sync_copy(zeros_hbm, buf_v)
  for j in range(RPT // CHUNK):
    pltpu.sync_copy(buf_v, deg_sh.at[pl.ds(sid * RPT + j * CHUNK, CHUNK)])
  pltpu.sync_copy(ones_hbm, ones_v)
  pltpu.sync_copy(dst_hbm.at[pl.ds(wid * KPT, KPT)], idx_v)
  plsc.subcore_barrier()

  def body(k, carry):
    pltpu.sync_copy(ones_v, deg_sh.at[idx_v.at[k]], add=True)
    return carry

  lax.fori_loop(0, KPT, body, 0)
  plsc.subcore_barrier()
  for j in range(RPT // CHUNK):
    off = sid * RPT + j * CHUNK
    pltpu.sync_copy(deg_sh.at[pl.ds(off, CHUNK)], buf_v)
    pltpu.sync_copy(buf_v, out_hbm.at[pl.ds(cid * ACC_ROWS + off, CHUNK)])


@functools.partial(
    pl.kernel,
    out_type=jax.ShapeDtypeStruct((NC * ACC_ROWS, D), jnp.float32),
    mesh=_MESH,
    scratch_types=[
        pltpu.VMEM((GSTAGE, GCHUNK), jnp.int32),
        pltpu.VMEM((GSTAGE, GCHUNK), jnp.int32),
        pltpu.VMEM((NBUF, GCHUNK, D), jnp.float32),
        pltpu.VMEM_SHARED((ACC_ROWS, D), jnp.float32),
        pltpu.SemaphoreType.DMA,
        pltpu.SemaphoreType.DMA,
    ],
)
def _sc_scatter(src_hbm, dst_hbm, hp_hbm, zeros_hbm, out_hbm,
                src_v, dst_v, rows_v, agg_sh, *sems):
  cid = lax.axis_index("c")
  sid = lax.axis_index("s")
  pltpu.sync_copy(zeros_hbm, rows_v.at[0])
  for j in range(RPT // GCHUNK):
    pltpu.sync_copy(rows_v.at[0],
                    agg_sh.at[pl.ds(sid * RPT + j * GCHUNK, GCHUNK)])
  plsc.subcore_barrier()

  tile_base = jnp.where(cid == 0, sid * F0, NS * F0 + sid * F1)
  nstages = jnp.where(cid == 0, F0 // GSTAGE, F1 // GSTAGE)

  def gather(k, buf):
    pltpu.async_copy(hp_hbm.at[src_v.at[k]], rows_v.at[buf], sems[buf])

  def gwait(buf):
    pltpu.make_async_copy(hp_hbm.at[src_v.at[0]], rows_v.at[buf],
                          sems[buf]).wait()

  def stage(h, carry):
    sb = tile_base + h * GSTAGE
    pltpu.sync_copy(src_hbm.at[pl.ds(sb, GSTAGE)], src_v)
    pltpu.sync_copy(dst_hbm.at[pl.ds(sb, GSTAGE)], dst_v)
    for b in range(NBUF):
      gather(b, b)

    def body(g, carry2):
      base = NBUF * g
      for b in range(NBUF):
        k = base + b
        gwait(b)
        pltpu.sync_copy(rows_v.at[b], agg_sh.at[dst_v.at[k]], add=True)
        gather(jnp.where(k + NBUF < GSTAGE, k + NBUF, 0), b)
      return carry2

    lax.fori_loop(0, GSTAGE // NBUF, body, 0)
    for b in range(NBUF):
      gwait(b)
    return carry

  lax.fori_loop(0, nstages, stage, 0)
  plsc.subcore_barrier()
  for j in range(RPT // GCHUNK):
    off = sid * RPT + j * GCHUNK
    pltpu.sync_copy(agg_sh.at[pl.ds(off, GCHUNK)], rows_v.at[0])
    pltpu.sync_copy(rows_v.at[0], out_hbm.at[pl.ds(cid * ACC_ROWS + off, GCHUNK)])


def _tc0_body(x_ref, w1_ref, h_ref):
  h_ref[...] = jnp.dot(x_ref[...], w1_ref[...],
                       preferred_element_type=jnp.float32)


def _tc1_body(h_ref, deg_ref, hp_ref, dinv_ref):
  d = deg_ref[...]
  cnt = d[0] + d[1] + 1.0
  dinv = lax.rsqrt(jnp.maximum(cnt, 1.0))
  dinv_ref[...] = dinv
  hp_ref[...] = h_ref[...] * dinv


def _tc2_body(agg_ref, hp_ref, dinv_ref, b1_ref, w2_ref, emb_ref, gp_ref):
  dinv = dinv_ref[...]
  agg = agg_ref[0] + agg_ref[1] + hp_ref[...]
  emb = jnp.maximum(agg * dinv + b1_ref[...], 0.0)
  emb_ref[...] = emb
  g = jnp.dot(emb, w2_ref[...], preferred_element_type=jnp.float32)
  gp_ref[...] = g * dinv


def _tc3_body(agg_ref, gp_ref, dinv_ref, b2_ref, out_ref):
  agg = agg_ref[0] + agg_ref[1] + gp_ref[...]
  out_ref[...] = agg * dinv_ref[...] + b2_ref[...]


_RB = 1000
_GRID = N // _RB

_agg_spec = pl.BlockSpec((NC, _RB, D), lambda i: (0, i, 0))
_row_spec = pl.BlockSpec((_RB, D), lambda i: (i, 0))
_w_spec = pl.BlockSpec((D, D), lambda i: (0, 0))
_b_spec = pl.BlockSpec((1, D), lambda i: (0, 0))


def kernel(x, edge_index, W1, b1, W2, b2):
  edge = edge_index.astype(jnp.int32)
  npad = E_PAD - E
  src2d = jnp.concatenate(
      [edge[0], jnp.zeros((npad,), jnp.int32)]).reshape(NCHUNK, CHUNK)
  dst2d = jnp.concatenate(
      [edge[1], jnp.full((npad,), N, jnp.int32)]).reshape(NCHUNK, CHUNK)
  src2dg = src2d.reshape(GTCH, GCHUNK)
  dst2dg = dst2d.reshape(GTCH, GCHUNK)
  ones128 = jnp.ones((CHUNK, D), jnp.float32)
  zeros128 = jnp.zeros((CHUNK, D), jnp.float32)
  b1r = b1.reshape(1, D)
  b2r = b2.reshape(1, D)

  deg = _sc_degree(dst2d, ones128, zeros128).reshape(NC, ACC_ROWS, D)

  h1 = pl.pallas_call(
      _tc0_body,
      grid=(_GRID,),
      in_specs=[_row_spec, _w_spec],
      out_specs=_row_spec,
      out_shape=jax.ShapeDtypeStruct((N, D), jnp.float32),
  )(x, W1)

  hp1, dinvb = pl.pallas_call(
      _tc1_body,
      grid=(_GRID,),
      in_specs=[_row_spec, _agg_spec],
      out_specs=[_row_spec, _row_spec],
      out_shape=[
          jax.ShapeDtypeStruct((N, D), jnp.float32),
          jax.ShapeDtypeStruct((N, D), jnp.float32),
      ],
  )(h1, deg)

  agg1 = _sc_scatter(src2dg, dst2dg, hp1, zeros128).reshape(NC, ACC_ROWS, D)

  emb, gp2 = pl.pallas_call(
      _tc2_body,
      grid=(_GRID,),
      in_specs=[_agg_spec, _row_spec, _row_spec, _b_spec, _w_spec],
      out_specs=[_row_spec, _row_spec],
      out_shape=[
          jax.ShapeDtypeStruct((N, D), jnp.float32),
          jax.ShapeDtypeStruct((N, D), jnp.float32),
      ],
  )(agg1, hp1, dinvb, b1r, W2)

  agg2 = _sc_scatter(src2dg, dst2dg, gp2, zeros128).reshape(NC, ACC_ROWS, D)

  out = pl.pallas_call(
      _tc3_body,
      grid=(_GRID,),
      in_specs=[_agg_spec, _row_spec, _row_spec, _b_spec],
      out_specs=_row_spec,
      out_shape=jax.ShapeDtypeStruct((N, D), jnp.float32),
  )(agg2, gp2, dinvb, b2r)

  return (emb, out)


# speedup vs baseline: 1.2828x; 1.2828x over previous
import functools

import jax
import jax.numpy as jnp
from jax import lax
from jax.experimental import pallas as pl
from jax.experimental.pallas import tpu as pltpu
from jax.experimental.pallas import tpu_sc as plsc

N = 10000
D = 128
E = 320000
NC = 2
NS = 16
NW = NC * NS
CHUNK = 128
KPT = 80
NCHUNK = NW * KPT
E_PAD = NCHUNK * CHUNK
GCHUNK = 64
GTCH = E_PAD // GCHUNK
NBUF = 4
GSTAGE = 32
F0 = 256
F1 = 64
assert NS * (F0 + F1) == GTCH
ACC_ROWS = 10240
RPT = ACC_ROWS // NS

_MESH = plsc.VectorSubcoreMesh(core_axis_name="c", subcore_axis_name="s")


@functools.partial(
    pl.kernel,
    out_type=jax.ShapeDtypeStruct((NC * ACC_ROWS, D), jnp.float32),
    mesh=_MESH,
    scratch_types=[
        pltpu.VMEM((KPT, CHUNK), jnp.int32),
        pltpu.VMEM((CHUNK, D), jnp.float32),
        pltpu.VMEM((CHUNK, D), jnp.float32),
        pltpu.VMEM_SHARED((ACC_ROWS, D), jnp.float32),
    ],
)
def _sc_degree(dst_hbm, ones_hbm, zeros_hbm, out_hbm,
               idx_v, ones_v, buf_v, deg_sh):
  cid = lax.axis_index("c")
  sid = lax.axis_index("s")
  wid = sid * NC + cid
  pltpu.sync_copy(zeros_hbm, buf_v)
  for j in range(RPT // CHUNK):
    pltpu.sync_copy(buf_v, deg_sh.at[pl.ds(sid * RPT + j * CHUNK, CHUNK)])
  pltpu.sync_copy(ones_hbm, ones_v)
  pltpu.sync_copy(dst_hbm.at[pl.ds(wid * KPT, KPT)], idx_v)
  plsc.subcore_barrier()

  def body(k, carry):
    pltpu.sync_copy(ones_v, deg_sh.at[idx_v.at[k]], add=True)
    return carry

  lax.fori_loop(0, KPT, body, 0)
  plsc.subcore_barrier()
  for j in range(RPT // CHUNK):
    off = sid * RPT + j * CHUNK
    pltpu.sync_copy(deg_sh.at[pl.ds(off, CHUNK)], buf_v)
    pltpu.sync_copy(buf_v, out_hbm.at[pl.ds(cid * ACC_ROWS + off, CHUNK)])


@functools.partial(
    pl.kernel,
    out_type=jax.ShapeDtypeStruct((NC * ACC_ROWS, D), jnp.float32),
    mesh=_MESH,
    scratch_types=[
        pltpu.VMEM((GSTAGE, GCHUNK), jnp.int32),
        pltpu.VMEM((GSTAGE, GCHUNK), jnp.int32),
        pltpu.VMEM((NBUF, GCHUNK, D), jnp.float32),
        pltpu.VMEM_SHARED((ACC_ROWS, D), jnp.float32),
        pltpu.SemaphoreType.DMA,
        pltpu.SemaphoreType.DMA,
        pltpu.SemaphoreType.DMA,
        pltpu.SemaphoreType.DMA,
    ],
)
def _sc_scatter(src_hbm, dst_hbm, hp_hbm, zeros_hbm, out_hbm,
                src_v, dst_v, rows_v, agg_sh, *sems):
  cid = lax.axis_index("c")
  sid = lax.axis_index("s")
  pltpu.sync_copy(zeros_hbm, rows_v.at[0])
  for j in range(RPT // GCHUNK):
    pltpu.sync_copy(rows_v.at[0],
                    agg_sh.at[pl.ds(sid * RPT + j * GCHUNK, GCHUNK)])
  plsc.subcore_barrier()

  tile_base = jnp.where(cid == 0, sid * F0, NS * F0 + sid * F1)
  nstages = jnp.where(cid == 0, F0 // GSTAGE, F1 // GSTAGE)

  def gather(k, buf):
    pltpu.async_copy(hp_hbm.at[src_v.at[k]], rows_v.at[buf], sems[buf])

  def gwait(buf):
    pltpu.make_async_copy(hp_hbm.at[src_v.at[0]], rows_v.at[buf],
                          sems[buf]).wait()

  def stage(h, carry):
    sb = tile_base + h * GSTAGE
    pltpu.sync_copy(src_hbm.at[pl.ds(sb, GSTAGE)], src_v)
    pltpu.sync_copy(dst_hbm.at[pl.ds(sb, GSTAGE)], dst_v)
    for b in range(NBUF):
      gather(b, b)

    def body(g, carry2):
      base = NBUF * g
      for b in range(NBUF):
        k = base + b
        gwait(b)
        pltpu.sync_copy(rows_v.at[b], agg_sh.at[dst_v.at[k]], add=True)
        gather(jnp.where(k + NBUF < GSTAGE, k + NBUF, 0), b)
      return carry2

    lax.fori_loop(0, GSTAGE // NBUF, body, 0)
    for b in range(NBUF):
      gwait(b)
    return carry

  lax.fori_loop(0, nstages, stage, 0)
  plsc.subcore_barrier()
  for j in range(RPT // GCHUNK):
    off = sid * RPT + j * GCHUNK
    pltpu.sync_copy(agg_sh.at[pl.ds(off, GCHUNK)], rows_v.at[0])
    pltpu.sync_copy(rows_v.at[0], out_hbm.at[pl.ds(cid * ACC_ROWS + off, GCHUNK)])


def _tc0_body(x_ref, w1_ref, h_ref):
  h_ref[...] = jnp.dot(x_ref[...], w1_ref[...],
                       preferred_element_type=jnp.float32)


def _tc1_body(h_ref, deg_ref, hp_ref, dinv_ref):
  d = deg_ref[...]
  cnt = d[0] + d[1] + 1.0
  dinv = lax.rsqrt(jnp.maximum(cnt, 1.0))
  dinv_ref[...] = dinv
  hp_ref[...] = h_ref[...] * dinv


def _tc2_body(agg_ref, hp_ref, dinv_ref, b1_ref, w2_ref, emb_ref, gp_ref):
  dinv = dinv_ref[...]
  agg = agg_ref[0] + agg_ref[1] + hp_ref[...]
  emb = jnp.maximum(agg * dinv + b1_ref[...], 0.0)
  emb_ref[...] = emb
  g = jnp.dot(emb, w2_ref[...], preferred_element_type=jnp.float32)
  gp_ref[...] = g * dinv


def _tc3_body(agg_ref, gp_ref, dinv_ref, b2_ref, out_ref):
  agg = agg_ref[0] + agg_ref[1] + gp_ref[...]
  out_ref[...] = agg * dinv_ref[...] + b2_ref[...]


_RB = 1000
_GRID = N // _RB

_agg_spec = pl.BlockSpec((NC, _RB, D), lambda i: (0, i, 0))
_row_spec = pl.BlockSpec((_RB, D), lambda i: (i, 0))
_w_spec = pl.BlockSpec((D, D), lambda i: (0, 0))
_b_spec = pl.BlockSpec((1, D), lambda i: (0, 0))


def kernel(x, edge_index, W1, b1, W2, b2):
  edge = edge_index.astype(jnp.int32)
  npad = E_PAD - E
  src2d = jnp.concatenate(
      [edge[0], jnp.zeros((npad,), jnp.int32)]).reshape(NCHUNK, CHUNK)
  dst2d = jnp.concatenate(
      [edge[1], jnp.full((npad,), N, jnp.int32)]).reshape(NCHUNK, CHUNK)
  src2dg = src2d.reshape(GTCH, GCHUNK)
  dst2dg = dst2d.reshape(GTCH, GCHUNK)
  ones128 = jnp.ones((CHUNK, D), jnp.float32)
  zeros128 = jnp.zeros((CHUNK, D), jnp.float32)
  zeros64 = jnp.zeros((GCHUNK, D), jnp.float32)
  b1r = b1.reshape(1, D)
  b2r = b2.reshape(1, D)

  deg = _sc_degree(dst2d, ones128, zeros128).reshape(NC, ACC_ROWS, D)

  h1 = pl.pallas_call(
      _tc0_body,
      grid=(_GRID,),
      in_specs=[_row_spec, _w_spec],
      out_specs=_row_spec,
      out_shape=jax.ShapeDtypeStruct((N, D), jnp.float32),
  )(x, W1)

  hp1, dinvb = pl.pallas_call(
      _tc1_body,
      grid=(_GRID,),
      in_specs=[_row_spec, _agg_spec],
      out_specs=[_row_spec, _row_spec],
      out_shape=[
          jax.ShapeDtypeStruct((N, D), jnp.float32),
          jax.ShapeDtypeStruct((N, D), jnp.float32),
      ],
  )(h1, deg)

  agg1 = _sc_scatter(src2dg, dst2dg, hp1, zeros64).reshape(NC, ACC_ROWS, D)

  emb, gp2 = pl.pallas_call(
      _tc2_body,
      grid=(_GRID,),
      in_specs=[_agg_spec, _row_spec, _row_spec, _b_spec, _w_spec],
      out_specs=[_row_spec, _row_spec],
      out_shape=[
          jax.ShapeDtypeStruct((N, D), jnp.float32),
          jax.ShapeDtypeStruct((N, D), jnp.float32),
      ],
  )(agg1, hp1, dinvb, b1r, W2)

  agg2 = _sc_scatter(src2dg, dst2dg, gp2, zeros64).reshape(NC, ACC_ROWS, D)

  out = pl.pallas_call(
      _tc3_body,
      grid=(_GRID,),
      in_specs=[_agg_spec, _row_spec, _row_spec, _b_spec],
      out_specs=_row_spec,
      out_shape=jax.ShapeDtypeStruct((N, D), jnp.float32),
  )(agg2, gp2, dinvb, b2r)

  return (emb, out)
